# baseline (speedup 1.0000x reference)
import numpy as np
import jax
import jax.numpy as jnp
from jax.experimental import pallas as pl
from jax.experimental.pallas import tpu as pltpu

_NB = 5
_EDGES = [[0, 1], [0, 2], [0, 3], [0, 4], [1, 0], [1, 2], [1, 3], [1, 4],
          [2, 0], [2, 1], [2, 3], [2, 4], [3, 0], [3, 1], [3, 2], [3, 4],
          [4, 0], [4, 1], [4, 2], [4, 3]]
_INCOMING = [[4, 8, 12, 16], [0, 9, 13, 17], [1, 5, 14, 18], [2, 6, 10, 19],
             [3, 7, 11, 15]]
_PRED_IDS = [[0, 10], [1, 11], [2, 12], [3, 13], [0, 14], [4, 15], [5, 16],
             [6, 17], [1, 18], [4, 19], [7, 20], [8, 21], [2, 22], [5, 23],
             [7, 24], [9, 25], [3, 26], [6, 27], [8, 28], [9, 29]]
_EDGE_ORDER = [e for inc in _INCOMING for e in inc]

_BT = 4096

_Z = 128


def _edge_onehot():
    P = np.zeros((20, _Z, 47), np.float32)
    for k, e in enumerate(_EDGE_ORDER):
        s, d = _EDGES[e]
        p0, p1 = _PRED_IDS[e]
        for i in range(10):
            P[k, i, i] = 1.0
        for i in range(4):
            P[k, 10 + i, 10 + i] = 1.0
        P[k, 14 + p0, 14] = 1.0
        P[k, 14 + p1, 15] = 1.0
        for i in range(15):
            P[k, 44 + 15 * s + i, 16 + i] = 1.0
            P[k, 44 + 15 * d + i, 31 + i] = 1.0
        P[k, 119, 46] = 1.0
    return P


def _phi_onehot():
    Q = np.zeros((_NB, _Z, 30), np.float32)
    for n in range(_NB):
        for i in range(4):
            Q[n, 10 + i, i] = 1.0
        for i in range(10):
            Q[n, i, 4 + i] = 1.0
        for i in range(15):
            Q[n, 44 + 15 * n + i, 14 + i] = 1.0
        Q[n, 119, 29] = 1.0
    return Q


_P_EDGE = _edge_onehot()
_Q_PHI = _phi_onehot()


def _dot(a, b):
    return jax.lax.dot_general(a, b, (((1,), (0,)), ((), ())),
                               preferred_element_type=jnp.float32)


def _fused_kernel(obs_ref, act_ref, ag_ref, g_ref,
                  We_ref, Wza_ref,
                  Wp1b_ref, bp1b_ref, Wp2b_ref, bp2b_ref,
                  Wr1a_ref, br1a_ref, wr1bT_ref, br1b_ref,
                  Wr2a_ref, br2a_ref, wr2bT_ref, br2b_ref,
                  q1_ref, q2_ref):
    obs = obs_ref[...]
    act = act_ref[...]
    dg = g_ref[...] - ag_ref[...]
    bt = obs.shape[0]
    z = jnp.concatenate(
        [obs[:, :10], act, dg, obs[:, 10:],
         jnp.full((bt, 1), 1.0, jnp.float32),
         jnp.zeros((bt, _Z - 120), jnp.float32)], axis=1)

    We = We_ref[...]
    Wza = Wza_ref[...]
    Wp1b = Wp1b_ref[...]
    Wp2b = Wp2b_ref[...]
    bp1b = bp1b_ref[...]
    bp2b = bp2b_ref[...]
    ef = jnp.maximum(_dot(z, We), 0.0)
    o1 = None
    o2 = None
    for n in range(_NB):
        y = ef[:, 512 * n:512 * n + 512]
        aggn = (y[:, 0:128] + y[:, 128:256] + y[:, 256:384] + y[:, 384:512])
        za = jnp.concatenate([z, aggn], axis=1)
        h = jnp.maximum(_dot(za, Wza[:, 512 * n:512 * n + 512]), 0.0)
        x1 = jnp.maximum(_dot(h[:, :256], Wp1b) + bp1b, 0.0)
        x2 = jnp.maximum(_dot(h[:, 256:], Wp2b) + bp2b, 0.0)
        o1 = x1 if o1 is None else o1 + x1
        o2 = x2 if o2 is None else o2 + x2

    u1 = jnp.maximum(_dot(o1, Wr1a_ref[...]) + br1a_ref[...], 0.0)
    q1_ref[...] = jnp.sum(u1 * wr1bT_ref[...], axis=1,
                          keepdims=True) + br1b_ref[...]
    u2 = jnp.maximum(_dot(o2, Wr2a_ref[...]) + br2a_ref[...], 0.0)
    q2_ref[...] = jnp.sum(u2 * wr2bT_ref[...], axis=1,
                          keepdims=True) + br2b_ref[...]


def kernel(obs, act, ag, g, W_mp, b_mp, W_p1a, b_p1a, W_p1b, b_p1b,
           W_p2a, b_p2a, W_p2b, b_p2b, W_r1a, b_r1a, W_r1b, b_r1b,
           W_r2a, b_r2a, W_r2b, b_r2b):
    batch = obs.shape[0]
    grid = (batch // _BT,)

    We_aug = jnp.concatenate([W_mp, b_mp[None]], axis=0)
    We = jnp.einsum('erj,jk->rek', _P_EDGE, We_aug).reshape(_Z, 20 * 128)
    W1_aug = jnp.concatenate([W_p1a[:29], b_p1a[None]], axis=0)
    W2_aug = jnp.concatenate([W_p2a[:29], b_p2a[None]], axis=0)
    Wz1 = jnp.einsum('nrj,jk->rnk', _Q_PHI, W1_aug)
    Wz2 = jnp.einsum('nrj,jk->rnk', _Q_PHI, W2_aug)
    Wz = jnp.concatenate([Wz1, Wz2], axis=2)
    Wagg = jnp.concatenate([W_p1a[29:], W_p2a[29:]], axis=1)
    Wza = jnp.concatenate(
        [Wz, jnp.broadcast_to(Wagg[:, None, :], (_Z, _NB, 512))],
        axis=0).reshape(2 * _Z, _NB * 512)

    args = (obs, act, ag, g, We, Wza,
            W_p1b, b_p1b.reshape(1, -1), W_p2b, b_p2b.reshape(1, -1),
            W_r1a, b_r1a.reshape(1, -1), W_r1b.reshape(1, -1),
            b_r1b.reshape(1, -1),
            W_r2a, b_r2a.reshape(1, -1), W_r2b.reshape(1, -1),
            b_r2b.reshape(1, -1))

    def row_spec(cols):
        return pl.BlockSpec((_BT, cols), lambda i: (i, 0))

    in_specs = [row_spec(obs.shape[1]), row_spec(act.shape[1]),
                row_spec(ag.shape[1]), row_spec(g.shape[1])]
    in_specs += [pl.BlockSpec(a.shape, lambda i: (0, 0)) for a in args[4:]]

    q1, q2 = pl.pallas_call(
        _fused_kernel,
        grid=grid,
        in_specs=in_specs,
        out_specs=[pl.BlockSpec((_BT, 1), lambda i: (i, 0))] * 2,
        out_shape=[jax.ShapeDtypeStruct((batch, 1), jnp.float32)] * 2,
        compiler_params=pltpu.CompilerParams(
            dimension_semantics=("parallel",)),
    )(*args)
    return (q1, q2)


# speedup vs baseline: 1.0065x; 1.0065x over previous
import numpy as np
import jax
import jax.numpy as jnp
from jax.experimental import pallas as pl
from jax.experimental.pallas import tpu as pltpu

_NB = 5
_EDGES = [[0, 1], [0, 2], [0, 3], [0, 4], [1, 0], [1, 2], [1, 3], [1, 4],
          [2, 0], [2, 1], [2, 3], [2, 4], [3, 0], [3, 1], [3, 2], [3, 4],
          [4, 0], [4, 1], [4, 2], [4, 3]]
_INCOMING = [[4, 8, 12, 16], [0, 9, 13, 17], [1, 5, 14, 18], [2, 6, 10, 19],
             [3, 7, 11, 15]]
_PRED_IDS = [[0, 10], [1, 11], [2, 12], [3, 13], [0, 14], [4, 15], [5, 16],
             [6, 17], [1, 18], [4, 19], [7, 20], [8, 21], [2, 22], [5, 23],
             [7, 24], [9, 25], [3, 26], [6, 27], [8, 28], [9, 29]]
_EDGE_ORDER = [e for inc in _INCOMING for e in inc]

_BT = 4096

_Z = 128


def _edge_onehot():
    P = np.zeros((20, _Z, 47), np.float32)
    for k, e in enumerate(_EDGE_ORDER):
        s, d = _EDGES[e]
        p0, p1 = _PRED_IDS[e]
        for i in range(10):
            P[k, i, i] = 1.0
        for i in range(4):
            P[k, 10 + i, 10 + i] = 1.0
        P[k, 14 + p0, 14] = 1.0
        P[k, 14 + p1, 15] = 1.0
        for i in range(15):
            P[k, 44 + 15 * s + i, 16 + i] = 1.0
            P[k, 44 + 15 * d + i, 31 + i] = 1.0
        P[k, 119, 46] = 1.0
    return P


def _phi_onehot():
    Q = np.zeros((_NB, _Z, 30), np.float32)
    for n in range(_NB):
        for i in range(4):
            Q[n, 10 + i, i] = 1.0
        for i in range(10):
            Q[n, i, 4 + i] = 1.0
        for i in range(15):
            Q[n, 44 + 15 * n + i, 14 + i] = 1.0
        Q[n, 119, 29] = 1.0
    return Q


_P_EDGE = _edge_onehot()
_Q_PHI = _phi_onehot()


def _dot(a, b):
    return jax.lax.dot_general(a.astype(jnp.bfloat16), b.astype(jnp.bfloat16),
                               (((1,), (0,)), ((), ())),
                               preferred_element_type=jnp.float32)


def _fused_kernel(obs_ref, act_ref, ag_ref, g_ref,
                  We_ref, Wza_ref,
                  Wp1b_ref, bp1b_ref, Wp2b_ref, bp2b_ref,
                  Wr1a_ref, br1a_ref, wr1bT_ref, br1b_ref,
                  Wr2a_ref, br2a_ref, wr2bT_ref, br2b_ref,
                  q1_ref, q2_ref):
    obs = obs_ref[...]
    act = act_ref[...]
    dg = g_ref[...] - ag_ref[...]
    bt = obs.shape[0]
    z = jnp.concatenate(
        [obs[:, :10], act, dg, obs[:, 10:],
         jnp.full((bt, 1), 1.0, jnp.float32),
         jnp.zeros((bt, _Z - 120), jnp.float32)], axis=1)

    We = We_ref[...]
    Wza = Wza_ref[...]
    Wp1b = Wp1b_ref[...]
    Wp2b = Wp2b_ref[...]
    bp1b = bp1b_ref[...]
    bp2b = bp2b_ref[...]
    o1 = None
    o2 = None
    for n in range(_NB):
        y = jnp.maximum(_dot(z, We[:, 512 * n:512 * n + 512]), 0.0)
        aggn = (y[:, 0:128] + y[:, 128:256] + y[:, 256:384] + y[:, 384:512])
        za = jnp.concatenate([z, aggn], axis=1)
        h = jnp.maximum(_dot(za, Wza[:, 512 * n:512 * n + 512]), 0.0)
        x1 = jnp.maximum(_dot(h[:, :256], Wp1b) + bp1b, 0.0)
        x2 = jnp.maximum(_dot(h[:, 256:], Wp2b) + bp2b, 0.0)
        o1 = x1 if o1 is None else o1 + x1
        o2 = x2 if o2 is None else o2 + x2

    u1 = jnp.maximum(_dot(o1, Wr1a_ref[...]) + br1a_ref[...], 0.0)
    q1_ref[...] = jnp.sum(u1 * wr1bT_ref[...], axis=1,
                          keepdims=True) + br1b_ref[...]
    u2 = jnp.maximum(_dot(o2, Wr2a_ref[...]) + br2a_ref[...], 0.0)
    q2_ref[...] = jnp.sum(u2 * wr2bT_ref[...], axis=1,
                          keepdims=True) + br2b_ref[...]


def kernel(obs, act, ag, g, W_mp, b_mp, W_p1a, b_p1a, W_p1b, b_p1b,
           W_p2a, b_p2a, W_p2b, b_p2b, W_r1a, b_r1a, W_r1b, b_r1b,
           W_r2a, b_r2a, W_r2b, b_r2b):
    batch = obs.shape[0]
    grid = (batch // _BT,)

    We_aug = jnp.concatenate([W_mp, b_mp[None]], axis=0)
    We = jnp.einsum('erj,jk->rek', _P_EDGE, We_aug).reshape(_Z, 20 * 128)
    W1_aug = jnp.concatenate([W_p1a[:29], b_p1a[None]], axis=0)
    W2_aug = jnp.concatenate([W_p2a[:29], b_p2a[None]], axis=0)
    Wz1 = jnp.einsum('nrj,jk->rnk', _Q_PHI, W1_aug)
    Wz2 = jnp.einsum('nrj,jk->rnk', _Q_PHI, W2_aug)
    Wz = jnp.concatenate([Wz1, Wz2], axis=2)
    Wagg = jnp.concatenate([W_p1a[29:], W_p2a[29:]], axis=1)
    Wza = jnp.concatenate(
        [Wz, jnp.broadcast_to(Wagg[:, None, :], (_Z, _NB, 512))],
        axis=0).reshape(2 * _Z, _NB * 512)

    args = (obs, act, ag, g, We, Wza,
            W_p1b, b_p1b.reshape(1, -1), W_p2b, b_p2b.reshape(1, -1),
            W_r1a, b_r1a.reshape(1, -1), W_r1b.reshape(1, -1),
            b_r1b.reshape(1, -1),
            W_r2a, b_r2a.reshape(1, -1), W_r2b.reshape(1, -1),
            b_r2b.reshape(1, -1))

    def row_spec(cols):
        return pl.BlockSpec((_BT, cols), lambda i: (i, 0))

    in_specs = [row_spec(obs.shape[1]), row_spec(act.shape[1]),
                row_spec(ag.shape[1]), row_spec(g.shape[1])]
    in_specs += [pl.BlockSpec(a.shape, lambda i: (0, 0)) for a in args[4:]]

    q1, q2 = pl.pallas_call(
        _fused_kernel,
        grid=grid,
        in_specs=in_specs,
        out_specs=[pl.BlockSpec((_BT, 1), lambda i: (i, 0))] * 2,
        out_shape=[jax.ShapeDtypeStruct((batch, 1), jnp.float32)] * 2,
        compiler_params=pltpu.CompilerParams(
            dimension_semantics=("parallel",)),
    )(*args)
    return (q1, q2)
